# baseline (speedup 1.0000x reference)
import jax
import jax.numpy as jnp
from jax.experimental import pallas as pl
from jax.experimental.pallas import tpu as pltpu

EMB = 1024
H = 256
B = 16
S = 20
N = B * S
T = 40
DOCF = 18
G4 = 4 * H
KT = 8
NI = T // KT


def _sig(x):
    return 0.5 * jnp.tanh(0.5 * x) + 0.5


def _lstm_update(gates, h_old, c_old, m):
    ig = _sig(gates[:, 0:H])
    fg = _sig(gates[:, H:2 * H])
    gg = jnp.tanh(gates[:, 2 * H:3 * H])
    og = _sig(gates[:, 3 * H:4 * H])
    c_new = fg * c_old + ig * gg
    h_new = og * jnp.tanh(c_new)
    out = m * h_new
    return out + (1.0 - m) * h_old, m * c_new + (1.0 - m) * c_old, out


NH = N // 2


def _rec0_body(slen_ref, len_ref, xf_ref, xb_ref, wif_ref, wib_ref,
               whf_ref, whb_ref, bf_ref, bb_ref, of_ref, ob_ref,
               hf, cf, hb, cb):
    i = pl.program_id(1)

    @pl.when(i == 0)
    def _init():
        z = jnp.zeros((NH, H), jnp.float32)
        hf[...] = z
        cf[...] = z
        hb[...] = z
        cb[...] = z

    lens = len_ref[...]
    maxlen = slen_ref[pl.program_id(0) * NH]

    @pl.when(maxlen > i * KT)
    def _fwd():
        for k in range(KT):
            t = i * KT + k
            m = (lens > t).astype(jnp.float32)
            gates = (xf_ref[:, k, :] @ wif_ref[...] + hf[...] @ whf_ref[...]
                     + bf_ref[...])
            hf[...], cf[...], out = _lstm_update(gates, hf[...], cf[...], m)
            of_ref[:, k, :] = out

    @pl.when(maxlen <= i * KT)
    def _fwd_zero():
        of_ref[...] = jnp.zeros((NH, KT, H), jnp.float32)

    tb0 = (NI - 1 - i) * KT

    @pl.when(maxlen > tb0)
    def _bwd():
        for k in range(KT - 1, -1, -1):
            t = tb0 + k
            m = (lens > t).astype(jnp.float32)
            gates = (xb_ref[:, k, :] @ wib_ref[...] + hb[...] @ whb_ref[...]
                     + bb_ref[...])
            hb[...], cb[...], out = _lstm_update(gates, hb[...], cb[...], m)
            ob_ref[:, k, :] = out

    @pl.when(maxlen <= tb0)
    def _bwd_zero():
        ob_ref[...] = jnp.zeros((NH, KT, H), jnp.float32)


def _rec0(lengths2d, x, WihTf, WihTb, WhhTf, WhhTb, bf, bb):
    return pl.pallas_call(
        _rec0_body,
        grid=(2, NI),
        in_specs=[
            pl.BlockSpec(memory_space=pltpu.SMEM),
            pl.BlockSpec((NH, 1), lambda nb, i: (nb, 0)),
            pl.BlockSpec((NH, KT, EMB), lambda nb, i: (nb, i, 0)),
            pl.BlockSpec((NH, KT, EMB), lambda nb, i: (nb, NI - 1 - i, 0)),
            pl.BlockSpec((EMB, G4), lambda nb, i: (0, 0)),
            pl.BlockSpec((EMB, G4), lambda nb, i: (0, 0)),
            pl.BlockSpec((H, G4), lambda nb, i: (0, 0)),
            pl.BlockSpec((H, G4), lambda nb, i: (0, 0)),
            pl.BlockSpec((1, G4), lambda nb, i: (0, 0)),
            pl.BlockSpec((1, G4), lambda nb, i: (0, 0)),
        ],
        out_specs=[
            pl.BlockSpec((NH, KT, H), lambda nb, i: (nb, i, 0)),
            pl.BlockSpec((NH, KT, H), lambda nb, i: (nb, NI - 1 - i, 0)),
        ],
        out_shape=[
            jax.ShapeDtypeStruct((N, T, H), jnp.float32),
            jax.ShapeDtypeStruct((N, T, H), jnp.float32),
        ],
        scratch_shapes=[
            pltpu.VMEM((NH, H), jnp.float32),
            pltpu.VMEM((NH, H), jnp.float32),
            pltpu.VMEM((NH, H), jnp.float32),
            pltpu.VMEM((NH, H), jnp.float32),
        ],
    )(lengths2d.reshape(N), lengths2d, x, x,
      WihTf, WihTb, WhhTf, WhhTb, bf, bb)


AC = 32


def _rec1a_body(len_ref, lrow_ref, ff_ref, fb_ref, rf_ref, rb_ref,
                wfa_ref, wfb_ref, wba_ref, wbb_ref, whf_ref, whb_ref,
                bf_ref, bb_ref, awf_ref, awb_ref, ab_ref, actx_ref,
                ef_ref, eb_ref, hf, cf, hb, cb, s1f, s1b):
    i = pl.program_id(0)

    @pl.when(i == 0)
    def _init():
        z = jnp.zeros((N, H), jnp.float32)
        hf[...] = z
        cf[...] = z
        hb[...] = z
        cb[...] = z

    lens = len_ref[...]
    for k in range(KT):
        t = i * KT + k
        m = (lens > t).astype(jnp.float32)
        gates = (ff_ref[:, k, :] @ wfa_ref[...]
                 + fb_ref[:, k, :] @ wfb_ref[...]
                 + hf[...] @ whf_ref[...] + bf_ref[...])
        hf[...], cf[...], out = _lstm_update(gates, hf[...], cf[...], m)
        s1f[t, :, :] = out.astype(jnp.bfloat16)
    for k in range(KT - 1, -1, -1):
        t = (NI - 1 - i) * KT + k
        m = (lens > t).astype(jnp.float32)
        gates = (rf_ref[:, k, :] @ wba_ref[...]
                 + rb_ref[:, k, :] @ wbb_ref[...]
                 + hb[...] @ whb_ref[...] + bb_ref[...])
        hb[...], cb[...], out = _lstm_update(gates, hb[...], cb[...], m)
        s1b[t, :, :] = out.astype(jnp.bfloat16)

    @pl.when(i == NI - 1)
    def _attn():
        lrow = lrow_ref[...]
        for c in range(N // AC):
            lo = c * AC
            hfc = s1f[:, lo:lo + AC, :].astype(jnp.float32)
            hbc = s1b[:, lo:lo + AC, :].astype(jnp.float32)
            u = jnp.tanh(hfc.reshape(T * AC, H) @ awf_ref[...]
                         + hbc.reshape(T * AC, H) @ awb_ref[...]
                         + ab_ref[...])
            s = (u @ actx_ref[...]).reshape(T, AC)
            mask = (jax.lax.broadcasted_iota(jnp.int32, (T, AC), 0)
                    < lrow[:, lo:lo + AC])
            s = jnp.where(mask, s, -1e9)
            smax = jnp.max(s, axis=0, keepdims=True)
            e = jnp.exp(s - smax)
            a = e / jnp.sum(e, axis=0, keepdims=True)
            a3 = a.reshape(T, AC, 1)
            ef_ref[lo:lo + AC, :] = jnp.sum(a3 * hfc, axis=0)
            eb_ref[lo:lo + AC, :] = jnp.sum(a3 * hbc, axis=0)


def _rec1attn(lengths2d, lengthsrow, o0f, o0b, WfA, WfB, WbA, WbB,
              WhhTf, WhhTb, bf, bb, aWf, aWb, ab, actx):
    hspec_f = pl.BlockSpec((N, KT, H), lambda i: (0, i, 0))
    hspec_r = pl.BlockSpec((N, KT, H), lambda i: (0, NI - 1 - i, 0))
    wspec = pl.BlockSpec((H, G4), lambda i: (0, 0))
    bspec = pl.BlockSpec((1, G4), lambda i: (0, 0))
    return pl.pallas_call(
        _rec1a_body,
        grid=(NI,),
        in_specs=[
            pl.BlockSpec((N, 1), lambda i: (0, 0)),
            pl.BlockSpec((1, N), lambda i: (0, 0)),
            hspec_f, hspec_f, hspec_r, hspec_r,
            wspec, wspec, wspec, wspec, wspec, wspec,
            bspec, bspec,
            pl.BlockSpec((H, 2 * H), lambda i: (0, 0)),
            pl.BlockSpec((H, 2 * H), lambda i: (0, 0)),
            pl.BlockSpec((1, 2 * H), lambda i: (0, 0)),
            pl.BlockSpec((2 * H, 1), lambda i: (0, 0)),
        ],
        out_specs=[
            pl.BlockSpec((N, H), lambda i: (0, 0)),
            pl.BlockSpec((N, H), lambda i: (0, 0)),
        ],
        out_shape=[
            jax.ShapeDtypeStruct((N, H), jnp.float32),
            jax.ShapeDtypeStruct((N, H), jnp.float32),
        ],
        scratch_shapes=[
            pltpu.VMEM((N, H), jnp.float32),
            pltpu.VMEM((N, H), jnp.float32),
            pltpu.VMEM((N, H), jnp.float32),
            pltpu.VMEM((N, H), jnp.float32),
            pltpu.VMEM((T, N, H), jnp.bfloat16),
            pltpu.VMEM((T, N, H), jnp.bfloat16),
        ],
    )(lengths2d, lengthsrow, o0f, o0b, o0f, o0b,
      WfA, WfB, WbA, WbB, WhhTf, WhhTb, bf, bb, aWf, aWb, ab, actx)


def _doc_body(idx_ref, ef_ref, eb_ref, wfa_ref, wfb_ref, wba_ref, wbb_ref,
              whf_ref, whb_ref, bf_ref, bb_ref, awf_ref, awb_ref, ab_ref,
              ctx_ref, df_ref, fcf_ref, fcb2_ref, fcd_ref, fcbias_ref,
              out_ref, gf, gb):
    def gather_one(j, _):
        r = idx_ref[j]
        d = j // S
        s = j - d * S
        k = s * B + d
        gf[k, :] = ef_ref[r, :]
        gb[k, :] = eb_ref[r, :]
        return 0

    jax.lax.fori_loop(0, N, gather_one, 0)

    bf = bf_ref[...]
    bb = bb_ref[...]

    def dstep(h, c, xf, xb, wa_ref, wb2_ref, wh_ref, bias):
        gates = xf @ wa_ref[...] + xb @ wb2_ref[...] + h @ wh_ref[...] + bias
        ig = jax.nn.sigmoid(gates[:, 0:H])
        fg = jax.nn.sigmoid(gates[:, H:2 * H])
        gg = jnp.tanh(gates[:, 2 * H:3 * H])
        og = jax.nn.sigmoid(gates[:, 3 * H:4 * H])
        c_new = fg * c + ig * gg
        h_new = og * jnp.tanh(c_new)
        return h_new, c_new

    z = jnp.zeros((B, H), jnp.float32)
    houts_f = [None] * S
    h, c = z, z
    for s in range(S):
        xf = gf[s * B:(s + 1) * B, :]
        xb = gb[s * B:(s + 1) * B, :]
        h, c = dstep(h, c, xf, xb, wfa_ref, wfb_ref, whf_ref, bf)
        houts_f[s] = h
    houts_b = [None] * S
    h, c = z, z
    for s in range(S - 1, -1, -1):
        xf = gf[s * B:(s + 1) * B, :]
        xb = gb[s * B:(s + 1) * B, :]
        h, c = dstep(h, c, xf, xb, wba_ref, wbb_ref, whb_ref, bb)
        houts_b[s] = h

    ab = ab_ref[...]
    ctx = ctx_ref[...]
    scores = [None] * S
    for s in range(S):
        u = jnp.tanh(houts_f[s] @ awf_ref[...] + houts_b[s] @ awb_ref[...] + ab)
        scores[s] = u @ ctx
    sc = jnp.concatenate(scores, axis=1)
    smax = jnp.max(sc, axis=1, keepdims=True)
    e = jnp.exp(sc - smax)
    a = e / jnp.sum(e, axis=1, keepdims=True)
    embf = jnp.zeros((B, H), jnp.float32)
    embb = jnp.zeros((B, H), jnp.float32)
    for s in range(S):
        embf = embf + a[:, s:s + 1] * houts_f[s]
        embb = embb + a[:, s:s + 1] * houts_b[s]

    logit = (embf @ fcf_ref[...] + embb @ fcb2_ref[...]
             + df_ref[...] @ fcd_ref[...] + fcbias_ref[...])
    out_ref[...] = jax.nn.sigmoid(logit)


def _doc_stage(recover_idx, ef, eb, dWfA, dWfB, dWbA, dWbB, dWhTf, dWhTb,
               dbf, dbb, aWf, aWb, abias, ctx, doc_features,
               fcf, fcb2, fcd, fcbias):
    wspec = pl.BlockSpec((H, G4), lambda: (0, 0))
    return pl.pallas_call(
        _doc_body,
        in_specs=[
            pl.BlockSpec(memory_space=pltpu.SMEM),
            pl.BlockSpec((N, H), lambda: (0, 0)),
            pl.BlockSpec((N, H), lambda: (0, 0)),
            wspec, wspec, wspec, wspec, wspec, wspec,
            pl.BlockSpec((1, G4), lambda: (0, 0)),
            pl.BlockSpec((1, G4), lambda: (0, 0)),
            pl.BlockSpec((H, 2 * H), lambda: (0, 0)),
            pl.BlockSpec((H, 2 * H), lambda: (0, 0)),
            pl.BlockSpec((1, 2 * H), lambda: (0, 0)),
            pl.BlockSpec((2 * H, 1), lambda: (0, 0)),
            pl.BlockSpec((B, DOCF), lambda: (0, 0)),
            pl.BlockSpec((H, 1), lambda: (0, 0)),
            pl.BlockSpec((H, 1), lambda: (0, 0)),
            pl.BlockSpec((DOCF, 1), lambda: (0, 0)),
            pl.BlockSpec((1, 1), lambda: (0, 0)),
        ],
        out_specs=pl.BlockSpec((B, 1), lambda: (0, 0)),
        out_shape=jax.ShapeDtypeStruct((B, 1), jnp.float32),
        scratch_shapes=[
            pltpu.VMEM((N, H), jnp.float32),
            pltpu.VMEM((N, H), jnp.float32),
        ],
    )(recover_idx, ef, eb, dWfA, dWfB, dWbA, dWbB, dWhTf, dWhTb,
      dbf, dbb, aWf, aWb, abias, ctx, doc_features,
      fcf, fcb2, fcd, fcbias)


def kernel(x, recover_idx, num_sent_per_document, sent_lengths, doc_features,
           s_Wih_0f, s_Whh_0f, s_b_0f, s_Wih_0b, s_Whh_0b, s_b_0b,
           s_Wih_1f, s_Whh_1f, s_b_1f, s_Wih_1b, s_Whh_1b, s_b_1b,
           s_attn_W, s_attn_b, s_attn_ctx,
           d_Wih_f, d_Whh_f, d_b_f, d_Wih_b, d_Whh_b, d_b_b,
           d_attn_W, d_attn_b, d_attn_ctx, fc_W, fc_b):
    lengths2d = sent_lengths.astype(jnp.int32).reshape(N, 1)

    o0f, o0b = _rec0(lengths2d, x, s_Wih_0f.T, s_Wih_0b.T,
                     s_Whh_0f.T, s_Whh_0b.T,
                     s_b_0f.reshape(1, G4), s_b_0b.reshape(1, G4))

    W1fT = s_Wih_1f.T
    W1bT = s_Wih_1b.T
    ef, eb = _rec1attn(lengths2d, lengths2d.reshape(1, N), o0f, o0b,
                       W1fT[:H], W1fT[H:], W1bT[:H], W1bT[H:],
                       s_Whh_1f.T, s_Whh_1b.T,
                       s_b_1f.reshape(1, G4), s_b_1b.reshape(1, G4),
                       s_attn_W[:H], s_attn_W[H:],
                       s_attn_b.reshape(1, 2 * H),
                       s_attn_ctx.reshape(2 * H, 1))

    dWfT = d_Wih_f.T
    dWbT = d_Wih_b.T
    out = _doc_stage(recover_idx.astype(jnp.int32), ef, eb,
                     dWfT[:H], dWfT[H:], dWbT[:H], dWbT[H:],
                     d_Whh_f.T, d_Whh_b.T,
                     d_b_f.reshape(1, G4), d_b_b.reshape(1, G4),
                     d_attn_W[:H], d_attn_W[H:],
                     d_attn_b.reshape(1, 2 * H),
                     d_attn_ctx.reshape(2 * H, 1),
                     doc_features,
                     fc_W[:H], fc_W[H:2 * H], fc_W[2 * H:], fc_b.reshape(1, 1))
    return out.reshape(-1)


# speedup vs baseline: 1.0035x; 1.0035x over previous
import jax
import jax.numpy as jnp
from jax.experimental import pallas as pl
from jax.experimental.pallas import tpu as pltpu

EMB = 1024
H = 256
B = 16
S = 20
N = B * S
T = 40
DOCF = 18
G4 = 4 * H
KT = 8
NI = T // KT


def _lstm_update(gates, h_old, c_old, m):
    ig = jax.nn.sigmoid(gates[:, 0:H])
    fg = jax.nn.sigmoid(gates[:, H:2 * H])
    gg = jnp.tanh(gates[:, 2 * H:3 * H])
    og = jax.nn.sigmoid(gates[:, 3 * H:4 * H])
    c_new = fg * c_old + ig * gg
    h_new = og * jnp.tanh(c_new)
    out = m * h_new
    return out + (1.0 - m) * h_old, m * c_new + (1.0 - m) * c_old, out


NH = N // 2


def _rec0_body(slen_ref, len_ref, xf_ref, xb_ref, wif_ref, wib_ref,
               whf_ref, whb_ref, bf_ref, bb_ref, of_ref, ob_ref,
               hf, cf, hb, cb):
    i = pl.program_id(1)

    @pl.when(i == 0)
    def _init():
        z = jnp.zeros((NH, H), jnp.float32)
        hf[...] = z
        cf[...] = z
        hb[...] = z
        cb[...] = z

    lens = len_ref[...]
    maxlen = slen_ref[pl.program_id(0) * NH]

    @pl.when(maxlen > i * KT)
    def _fwd():
        for k in range(KT):
            t = i * KT + k
            m = (lens > t).astype(jnp.float32)
            gates = (xf_ref[:, k, :] @ wif_ref[...] + hf[...] @ whf_ref[...]
                     + bf_ref[...])
            hf[...], cf[...], out = _lstm_update(gates, hf[...], cf[...], m)
            of_ref[:, k, :] = out

    @pl.when(maxlen <= i * KT)
    def _fwd_zero():
        of_ref[...] = jnp.zeros((NH, KT, H), jnp.float32)

    tb0 = (NI - 1 - i) * KT

    @pl.when(maxlen > tb0)
    def _bwd():
        for k in range(KT - 1, -1, -1):
            t = tb0 + k
            m = (lens > t).astype(jnp.float32)
            gates = (xb_ref[:, k, :] @ wib_ref[...] + hb[...] @ whb_ref[...]
                     + bb_ref[...])
            hb[...], cb[...], out = _lstm_update(gates, hb[...], cb[...], m)
            ob_ref[:, k, :] = out

    @pl.when(maxlen <= tb0)
    def _bwd_zero():
        ob_ref[...] = jnp.zeros((NH, KT, H), jnp.float32)


def _rec0(lengths2d, x, WihTf, WihTb, WhhTf, WhhTb, bf, bb):
    return pl.pallas_call(
        _rec0_body,
        grid=(2, NI),
        in_specs=[
            pl.BlockSpec(memory_space=pltpu.SMEM),
            pl.BlockSpec((NH, 1), lambda nb, i: (nb, 0)),
            pl.BlockSpec((NH, KT, EMB), lambda nb, i: (nb, i, 0)),
            pl.BlockSpec((NH, KT, EMB), lambda nb, i: (nb, NI - 1 - i, 0)),
            pl.BlockSpec((EMB, G4), lambda nb, i: (0, 0)),
            pl.BlockSpec((EMB, G4), lambda nb, i: (0, 0)),
            pl.BlockSpec((H, G4), lambda nb, i: (0, 0)),
            pl.BlockSpec((H, G4), lambda nb, i: (0, 0)),
            pl.BlockSpec((1, G4), lambda nb, i: (0, 0)),
            pl.BlockSpec((1, G4), lambda nb, i: (0, 0)),
        ],
        out_specs=[
            pl.BlockSpec((NH, KT, H), lambda nb, i: (nb, i, 0)),
            pl.BlockSpec((NH, KT, H), lambda nb, i: (nb, NI - 1 - i, 0)),
        ],
        out_shape=[
            jax.ShapeDtypeStruct((N, T, H), jnp.float32),
            jax.ShapeDtypeStruct((N, T, H), jnp.float32),
        ],
        scratch_shapes=[
            pltpu.VMEM((NH, H), jnp.float32),
            pltpu.VMEM((NH, H), jnp.float32),
            pltpu.VMEM((NH, H), jnp.float32),
            pltpu.VMEM((NH, H), jnp.float32),
        ],
    )(lengths2d.reshape(N), lengths2d, x, x,
      WihTf, WihTb, WhhTf, WhhTb, bf, bb)


AC = 32


def _rec1a_body(len_ref, lrow_ref, ff_ref, fb_ref, rf_ref, rb_ref,
                wfa_ref, wfb_ref, wba_ref, wbb_ref, whf_ref, whb_ref,
                bf_ref, bb_ref, awf_ref, awb_ref, ab_ref, actx_ref,
                ef_ref, eb_ref, hf, cf, hb, cb, s1f, s1b):
    i = pl.program_id(0)

    @pl.when(i == 0)
    def _init():
        z = jnp.zeros((N, H), jnp.float32)
        hf[...] = z
        cf[...] = z
        hb[...] = z
        cb[...] = z

    lens = len_ref[...]
    for k in range(KT):
        t = i * KT + k
        m = (lens > t).astype(jnp.float32)
        gates = (ff_ref[:, k, :] @ wfa_ref[...]
                 + fb_ref[:, k, :] @ wfb_ref[...]
                 + hf[...] @ whf_ref[...] + bf_ref[...])
        hf[...], cf[...], out = _lstm_update(gates, hf[...], cf[...], m)
        s1f[t, :, :] = out.astype(jnp.bfloat16)
    for k in range(KT - 1, -1, -1):
        t = (NI - 1 - i) * KT + k
        m = (lens > t).astype(jnp.float32)
        gates = (rf_ref[:, k, :] @ wba_ref[...]
                 + rb_ref[:, k, :] @ wbb_ref[...]
                 + hb[...] @ whb_ref[...] + bb_ref[...])
        hb[...], cb[...], out = _lstm_update(gates, hb[...], cb[...], m)
        s1b[t, :, :] = out.astype(jnp.bfloat16)

    @pl.when(i == NI - 1)
    def _attn():
        lrow = lrow_ref[...]
        for c in range(N // AC):
            lo = c * AC
            hfc = s1f[:, lo:lo + AC, :].astype(jnp.float32)
            hbc = s1b[:, lo:lo + AC, :].astype(jnp.float32)
            u = jnp.tanh(hfc.reshape(T * AC, H) @ awf_ref[...]
                         + hbc.reshape(T * AC, H) @ awb_ref[...]
                         + ab_ref[...])
            s = (u @ actx_ref[...]).reshape(T, AC)
            mask = (jax.lax.broadcasted_iota(jnp.int32, (T, AC), 0)
                    < lrow[:, lo:lo + AC])
            s = jnp.where(mask, s, -1e9)
            smax = jnp.max(s, axis=0, keepdims=True)
            e = jnp.exp(s - smax)
            a = e / jnp.sum(e, axis=0, keepdims=True)
            a3 = a.reshape(T, AC, 1)
            ef_ref[lo:lo + AC, :] = jnp.sum(a3 * hfc, axis=0)
            eb_ref[lo:lo + AC, :] = jnp.sum(a3 * hbc, axis=0)


def _rec1attn(lengths2d, lengthsrow, o0f, o0b, WfA, WfB, WbA, WbB,
              WhhTf, WhhTb, bf, bb, aWf, aWb, ab, actx):
    hspec_f = pl.BlockSpec((N, KT, H), lambda i: (0, i, 0))
    hspec_r = pl.BlockSpec((N, KT, H), lambda i: (0, NI - 1 - i, 0))
    wspec = pl.BlockSpec((H, G4), lambda i: (0, 0))
    bspec = pl.BlockSpec((1, G4), lambda i: (0, 0))
    return pl.pallas_call(
        _rec1a_body,
        grid=(NI,),
        in_specs=[
            pl.BlockSpec((N, 1), lambda i: (0, 0)),
            pl.BlockSpec((1, N), lambda i: (0, 0)),
            hspec_f, hspec_f, hspec_r, hspec_r,
            wspec, wspec, wspec, wspec, wspec, wspec,
            bspec, bspec,
            pl.BlockSpec((H, 2 * H), lambda i: (0, 0)),
            pl.BlockSpec((H, 2 * H), lambda i: (0, 0)),
            pl.BlockSpec((1, 2 * H), lambda i: (0, 0)),
            pl.BlockSpec((2 * H, 1), lambda i: (0, 0)),
        ],
        out_specs=[
            pl.BlockSpec((N, H), lambda i: (0, 0)),
            pl.BlockSpec((N, H), lambda i: (0, 0)),
        ],
        out_shape=[
            jax.ShapeDtypeStruct((N, H), jnp.float32),
            jax.ShapeDtypeStruct((N, H), jnp.float32),
        ],
        scratch_shapes=[
            pltpu.VMEM((N, H), jnp.float32),
            pltpu.VMEM((N, H), jnp.float32),
            pltpu.VMEM((N, H), jnp.float32),
            pltpu.VMEM((N, H), jnp.float32),
            pltpu.VMEM((T, N, H), jnp.bfloat16),
            pltpu.VMEM((T, N, H), jnp.bfloat16),
        ],
    )(lengths2d, lengthsrow, o0f, o0b, o0f, o0b,
      WfA, WfB, WbA, WbB, WhhTf, WhhTb, bf, bb, aWf, aWb, ab, actx)


def _doc_body(idx_ref, ef_ref, eb_ref, wfa_ref, wfb_ref, wba_ref, wbb_ref,
              whf_ref, whb_ref, bf_ref, bb_ref, awf_ref, awb_ref, ab_ref,
              ctx_ref, df_ref, fcf_ref, fcb2_ref, fcd_ref, fcbias_ref,
              out_ref, gf, gb):
    def gather_one(j, _):
        r = idx_ref[j]
        d = j // S
        s = j - d * S
        k = s * B + d
        gf[k, :] = ef_ref[r, :]
        gb[k, :] = eb_ref[r, :]
        return 0

    jax.lax.fori_loop(0, N, gather_one, 0)

    bf = bf_ref[...]
    bb = bb_ref[...]

    def dstep(h, c, xf, xb, wa_ref, wb2_ref, wh_ref, bias):
        gates = xf @ wa_ref[...] + xb @ wb2_ref[...] + h @ wh_ref[...] + bias
        ig = jax.nn.sigmoid(gates[:, 0:H])
        fg = jax.nn.sigmoid(gates[:, H:2 * H])
        gg = jnp.tanh(gates[:, 2 * H:3 * H])
        og = jax.nn.sigmoid(gates[:, 3 * H:4 * H])
        c_new = fg * c + ig * gg
        h_new = og * jnp.tanh(c_new)
        return h_new, c_new

    z = jnp.zeros((B, H), jnp.float32)
    houts_f = [None] * S
    h, c = z, z
    for s in range(S):
        xf = gf[s * B:(s + 1) * B, :]
        xb = gb[s * B:(s + 1) * B, :]
        h, c = dstep(h, c, xf, xb, wfa_ref, wfb_ref, whf_ref, bf)
        houts_f[s] = h
    houts_b = [None] * S
    h, c = z, z
    for s in range(S - 1, -1, -1):
        xf = gf[s * B:(s + 1) * B, :]
        xb = gb[s * B:(s + 1) * B, :]
        h, c = dstep(h, c, xf, xb, wba_ref, wbb_ref, whb_ref, bb)
        houts_b[s] = h

    ab = ab_ref[...]
    ctx = ctx_ref[...]
    scores = [None] * S
    for s in range(S):
        u = jnp.tanh(houts_f[s] @ awf_ref[...] + houts_b[s] @ awb_ref[...] + ab)
        scores[s] = u @ ctx
    sc = jnp.concatenate(scores, axis=1)
    smax = jnp.max(sc, axis=1, keepdims=True)
    e = jnp.exp(sc - smax)
    a = e / jnp.sum(e, axis=1, keepdims=True)
    embf = jnp.zeros((B, H), jnp.float32)
    embb = jnp.zeros((B, H), jnp.float32)
    for s in range(S):
        embf = embf + a[:, s:s + 1] * houts_f[s]
        embb = embb + a[:, s:s + 1] * houts_b[s]

    logit = (embf @ fcf_ref[...] + embb @ fcb2_ref[...]
             + df_ref[...] @ fcd_ref[...] + fcbias_ref[...])
    out_ref[...] = jax.nn.sigmoid(logit)


def _doc_stage(recover_idx, ef, eb, dWfA, dWfB, dWbA, dWbB, dWhTf, dWhTb,
               dbf, dbb, aWf, aWb, abias, ctx, doc_features,
               fcf, fcb2, fcd, fcbias):
    wspec = pl.BlockSpec((H, G4), lambda: (0, 0))
    return pl.pallas_call(
        _doc_body,
        in_specs=[
            pl.BlockSpec(memory_space=pltpu.SMEM),
            pl.BlockSpec((N, H), lambda: (0, 0)),
            pl.BlockSpec((N, H), lambda: (0, 0)),
            wspec, wspec, wspec, wspec, wspec, wspec,
            pl.BlockSpec((1, G4), lambda: (0, 0)),
            pl.BlockSpec((1, G4), lambda: (0, 0)),
            pl.BlockSpec((H, 2 * H), lambda: (0, 0)),
            pl.BlockSpec((H, 2 * H), lambda: (0, 0)),
            pl.BlockSpec((1, 2 * H), lambda: (0, 0)),
            pl.BlockSpec((2 * H, 1), lambda: (0, 0)),
            pl.BlockSpec((B, DOCF), lambda: (0, 0)),
            pl.BlockSpec((H, 1), lambda: (0, 0)),
            pl.BlockSpec((H, 1), lambda: (0, 0)),
            pl.BlockSpec((DOCF, 1), lambda: (0, 0)),
            pl.BlockSpec((1, 1), lambda: (0, 0)),
        ],
        out_specs=pl.BlockSpec((B, 1), lambda: (0, 0)),
        out_shape=jax.ShapeDtypeStruct((B, 1), jnp.float32),
        scratch_shapes=[
            pltpu.VMEM((N, H), jnp.float32),
            pltpu.VMEM((N, H), jnp.float32),
        ],
    )(recover_idx, ef, eb, dWfA, dWfB, dWbA, dWbB, dWhTf, dWhTb,
      dbf, dbb, aWf, aWb, abias, ctx, doc_features,
      fcf, fcb2, fcd, fcbias)


def kernel(x, recover_idx, num_sent_per_document, sent_lengths, doc_features,
           s_Wih_0f, s_Whh_0f, s_b_0f, s_Wih_0b, s_Whh_0b, s_b_0b,
           s_Wih_1f, s_Whh_1f, s_b_1f, s_Wih_1b, s_Whh_1b, s_b_1b,
           s_attn_W, s_attn_b, s_attn_ctx,
           d_Wih_f, d_Whh_f, d_b_f, d_Wih_b, d_Whh_b, d_b_b,
           d_attn_W, d_attn_b, d_attn_ctx, fc_W, fc_b):
    lengths2d = sent_lengths.astype(jnp.int32).reshape(N, 1)

    o0f, o0b = _rec0(lengths2d, x, s_Wih_0f.T, s_Wih_0b.T,
                     s_Whh_0f.T, s_Whh_0b.T,
                     s_b_0f.reshape(1, G4), s_b_0b.reshape(1, G4))

    W1fT = s_Wih_1f.T
    W1bT = s_Wih_1b.T
    ef, eb = _rec1attn(lengths2d, lengths2d.reshape(1, N), o0f, o0b,
                       W1fT[:H], W1fT[H:], W1bT[:H], W1bT[H:],
                       s_Whh_1f.T, s_Whh_1b.T,
                       s_b_1f.reshape(1, G4), s_b_1b.reshape(1, G4),
                       s_attn_W[:H], s_attn_W[H:],
                       s_attn_b.reshape(1, 2 * H),
                       s_attn_ctx.reshape(2 * H, 1))

    dWfT = d_Wih_f.T
    dWbT = d_Wih_b.T
    out = _doc_stage(recover_idx.astype(jnp.int32), ef, eb,
                     dWfT[:H], dWfT[H:], dWbT[:H], dWbT[H:],
                     d_Whh_f.T, d_Whh_b.T,
                     d_b_f.reshape(1, G4), d_b_b.reshape(1, G4),
                     d_attn_W[:H], d_attn_W[H:],
                     d_attn_b.reshape(1, 2 * H),
                     d_attn_ctx.reshape(2 * H, 1),
                     doc_features,
                     fc_W[:H], fc_W[H:2 * H], fc_W[2 * H:], fc_b.reshape(1, 1))
    return out.reshape(-1)
